# baseline (speedup 1.0000x reference)
import functools

import jax
import jax.numpy as jnp
from jax.experimental import pallas as pl
from jax.experimental.pallas import tpu as pltpu

_G = 8


def _shift_w(x):
    return jnp.concatenate(
        [x[..., 1:, :], jnp.zeros_like(x[..., :1, :])], axis=-2)


def _next_group(main0, halo0):
    return jnp.concatenate([main0[1:], halo0], axis=0)


def _pool_w_pair(parts):
    kw = len(parts)
    out = []
    for tw in range(kw // 2):
        col = []
        for th in range(len(parts[0])):
            nxt = (parts[2 * tw + 2][th] if 2 * tw + 2 < kw
                   else _shift_w(parts[0][th]))
            col.append(parts[2 * tw][th] + parts[2 * tw + 1][th] + nxt)
        out.append(col)
    return out


def _pool_h_pair(pm, ph):
    kh = len(pm[0])
    om, oh = [], []
    for tw in range(len(pm)):
        cm, ch = [], []
        for th in range(kh // 2):
            if 2 * th + 2 < kh:
                nm, nh = pm[tw][2 * th + 2], ph[tw][2 * th + 2]
            else:
                nm = _next_group(pm[tw][0], ph[tw][0])
                nh = jnp.zeros_like(ph[tw][0])
            cm.append(pm[tw][2 * th] + pm[tw][2 * th + 1] + nm)
            ch.append(ph[tw][2 * th] + ph[tw][2 * th + 1] + nh)
        om.append(cm)
        oh.append(ch)
    return om, oh


def _fused_kernel(n_tiles, x_ref, halo_ref, out2_ref, out3_ref):
    i = pl.program_id(0)
    halo = halo_ref[0]
    halo = jnp.where(i == n_tiles - 1, jnp.zeros_like(halo), halo)
    xm = x_ref[0]

    hm, hh = [], []
    for th in range(4):
        if 2 * th + 2 < 8:
            nm, nh = xm[:, 2 * th + 2], halo[:, 2 * th + 2]
        else:
            nm = _next_group(xm[:, 0], halo[:, 0])
            nh = jnp.zeros_like(halo[:, 0])
        hm.append(xm[:, 2 * th] + xm[:, 2 * th + 1] + nm)
        hh.append(halo[:, 2 * th] + halo[:, 2 * th + 1] + nh)

    tm = [jnp.swapaxes(hm[th], 1, 2) for th in range(4)]
    th_ = [jnp.swapaxes(hh[th], 1, 2) for th in range(4)]
    pm = [[tm[th][:, tw] for th in range(4)] for tw in range(8)]
    ph = [[th_[th][:, tw] for th in range(4)] for tw in range(8)]

    c1m, c1h = _pool_w_pair(pm), _pool_w_pair(ph)
    c2m, c2h = _pool_h_pair(_pool_w_pair(c1m), _pool_w_pair(c1h))

    out2_ref[0] = jnp.stack(
        [jnp.concatenate([c2m[0][th], c2m[1][th]], axis=-1)
         for th in range(2)], axis=1)

    c3wm = [c2m[0][th] + c2m[1][th] + _shift_w(c2m[0][th]) for th in range(2)]
    c3wh0 = c2h[0][0] + c2h[1][0] + _shift_w(c2h[0][0])
    out3_ref[0] = (c3wm[0] + c3wm[1] + _next_group(c3wm[0], c3wh0))


@jax.jit
def kernel(input):
    b, h, w, c = input.shape
    assert h % (8 * _G) == 0 and w % 8 == 0
    hg, wg = h // 8, w // 8
    n_tiles = hg // _G

    grid = (n_tiles,)

    in_spec = pl.BlockSpec((1, _G, 8, wg, 8, c), lambda i: (0, i, 0, 0, 0, 0))
    halo_spec = pl.BlockSpec(
        (1, 1, 8, wg, 8, c),
        lambda i: (0, jnp.minimum((i + 1) * _G, hg - 1), 0, 0, 0, 0))
    out2_spec = pl.BlockSpec((1, _G, 2, wg, 2 * c), lambda i: (0, i, 0, 0, 0))
    out3_spec = pl.BlockSpec((1, _G, wg, c), lambda i: (0, i, 0, 0))

    call = pl.pallas_call(
        functools.partial(_fused_kernel, n_tiles),
        grid=grid,
        in_specs=[in_spec, halo_spec],
        out_specs=[out2_spec, out3_spec],
        out_shape=[
            jax.ShapeDtypeStruct((1, hg, 2, wg, 2 * c), input.dtype),
            jax.ShapeDtypeStruct((1, hg, wg, c), input.dtype),
        ],
        compiler_params=pltpu.CompilerParams(
            dimension_semantics=("arbitrary",)),
    )
    o2s, o3s = [], []
    for bi in range(b):
        xb = input[bi:bi + 1].reshape(1, hg, 8, wg, 8, c)
        o2b, o3b = call(xb, xb)
        o2s.append(o2b)
        o3s.append(o3b)
    out2 = jnp.concatenate(o2s, axis=0)
    out3 = jnp.concatenate(o3s, axis=0)
    return out2.reshape(b, h // 4, w // 4, c), out3


# speedup vs baseline: 1.4718x; 1.4718x over previous
import functools

import jax
import jax.numpy as jnp
from jax.experimental import pallas as pl
from jax.experimental.pallas import tpu as pltpu

_G = 8


def _shift_w(x):
    return jnp.concatenate(
        [x[..., 1:, :], jnp.zeros_like(x[..., :1, :])], axis=-2)


def _shift_rows(x):
    return jnp.concatenate([x[1:], jnp.zeros_like(x[:1])], axis=0)


def _pool_w_list(parts):
    kw = len(parts)
    return [parts[2 * t] + parts[2 * t + 1] +
            (parts[2 * t + 2] if 2 * t + 2 < kw else _shift_w(parts[0]))
            for t in range(kw // 2)]


def _pool_w_grid(parts):
    kw = len(parts)
    out = []
    for tw in range(kw // 2):
        out.append([
            parts[2 * tw][th] + parts[2 * tw + 1][th] +
            (parts[2 * tw + 2][th] if 2 * tw + 2 < kw
             else _shift_w(parts[0][th]))
            for th in range(len(parts[0]))
        ])
    return out


def _pool_h_grid(parts):
    kh = len(parts[0])
    out = []
    for tw in range(len(parts)):
        col = []
        for th in range(kh // 2):
            nxt = (parts[tw][2 * th + 2] if 2 * th + 2 < kh
                   else _shift_rows(parts[tw][0]))
            col.append(parts[tw][2 * th] + parts[tw][2 * th + 1] + nxt)
        out.append(col)
    return out


def _kernel(n_tiles, x_ref, out2_ref, out3_ref,
            c1_ref, c2_ref, o2_ref, o3_ref):
    s = pl.program_id(0)
    g_ = _G
    _, rows, w, c = x_ref.shape
    wg = w // 8
    xm = x_ref[0].reshape(rows // 8, 8, wg, 8, c)

    hm = []
    for th in range(4):
        nxt = (xm[:, 2 * th + 2] if 2 * th + 2 < 8
               else _shift_rows(xm[:, 0]))
        hm.append(xm[:, 2 * th] + xm[:, 2 * th + 1] + nxt)
    tm = [jnp.swapaxes(h, 1, 2) for h in hm]
    pm = [[tm[th][:, tw] for th in range(4)] for tw in range(8)]
    c1m = _pool_w_grid(pm)
    c2m = _pool_h_grid(_pool_w_grid(c1m))

    gate = jnp.where(s % n_tiles == 0, 0.0, 1.0)
    old_a = [c1_ref[t] for t in range(4)]
    old_b = [c1_ref[4 + t] for t in range(4)]
    old_c2 = [c2_ref[t] for t in range(2)]

    r0 = [xm[0, 0, :, tw, :] for tw in range(8)]
    r0w = _pool_w_list(r0)
    c1_b = [old_b[t] + gate * r0w[t] for t in range(4)]
    c1n = [[gate * c1m[t][k][0] for t in range(4)]
           for k in range(3)]

    h_a = [old_a[t] + c1_b[t] + c1n[0][t] for t in range(4)]
    c2_b = _pool_w_list(h_a)
    h_n = [c1n[0][t] + c1n[1][t] + c1n[2][t] for t in range(4)]
    c2_n = _pool_w_list(h_n)

    o2_ref[g_ - 1, 1] = jnp.concatenate([c2_b[0], c2_b[1]], axis=-1)
    out2_ref[0] = o2_ref[...]

    c3 = sum(r[0] + r[1] + _shift_w(r[0])
             for r in (old_c2, c2_b, c2_n))
    o3_ref[g_ - 1] = c3
    out3_ref[0] = o3_ref[...]

    for t in range(4):
        c1_ref[t] = c1m[t][2][g_ - 1]
        c1_ref[4 + t] = c1m[t][3][g_ - 1]
    for t in range(2):
        c2_ref[t] = c2m[t][0][g_ - 1]
    for th in range(2):
        o2_ref[:, th] = jnp.concatenate([c2m[0][th], c2m[1][th]], axis=-1)
    c3w = [c2m[0][th] + c2m[1][th] + _shift_w(c2m[0][th]) for th in range(2)]
    o3_ref[...] = c3w[0] + c3w[1] + _shift_rows(c3w[0])


@jax.jit
def kernel(input):
    b, h, w, c = input.shape
    assert h % (8 * _G) == 0 and w % 8 == 0
    hg, wg = h // 8, w // 8
    n_tiles = hg // _G
    nb = b * n_tiles

    def in_idx(s):
        j = jnp.minimum(s, nb - 1)
        return (j // n_tiles, j % n_tiles, 0, 0)

    def out_idx2(s):
        t = jnp.maximum(s - 1, 0)
        return (t // n_tiles, t % n_tiles, 0, 0, 0)

    def out_idx3(s):
        t = jnp.maximum(s - 1, 0)
        return (t // n_tiles, t % n_tiles, 0, 0)

    out2, out3 = pl.pallas_call(
        functools.partial(_kernel, n_tiles),
        grid=(nb + 1,),
        in_specs=[pl.BlockSpec((1, 8 * _G, w, c), in_idx)],
        out_specs=[
            pl.BlockSpec((1, _G, 2, wg, 2 * c), out_idx2),
            pl.BlockSpec((1, _G, wg, c), out_idx3),
        ],
        out_shape=[
            jax.ShapeDtypeStruct((b, hg, 2, wg, 2 * c), input.dtype),
            jax.ShapeDtypeStruct((b, hg, wg, c), input.dtype),
        ],
        scratch_shapes=[
            pltpu.VMEM((8, wg, c), input.dtype),
            pltpu.VMEM((2, wg, c), input.dtype),
            pltpu.VMEM((_G, 2, wg, 2 * c), input.dtype),
            pltpu.VMEM((_G, wg, c), input.dtype),
        ],
        compiler_params=pltpu.CompilerParams(
            dimension_semantics=("arbitrary",)),
    )(input)
    return out2.reshape(b, h // 4, w // 4, c), out3
